# Initial kernel scaffold; baseline (speedup 1.0000x reference)
import jax
import jax.numpy as jnp
from jax.experimental import pallas as pl


def kernel(target, context, negative_samples, emb, ctx_emb):
    raise NotImplementedError("write your pallas kernel here")


# speedup vs baseline: 5.3699x; 5.3699x over previous
import functools

import jax
import jax.numpy as jnp
from jax import lax
from jax.experimental import pallas as pl
from jax.experimental.pallas import tpu as pltpu
from jax.experimental.pallas import tpu_sc as plsc

_V = 1000000
_D = 64
_B = 16384
_K = 20

_NC, _NS = 2, 16
_NW = _NC * _NS
_BW = _B // _NW
_CB = 32
_NCHUNK = _BW // _CB
_GI = 128
_NEG_I = _CB * _K
_NEG_G = _NEG_I // _GI


def _sc_body(tgt_hbm, ctx_hbm, neg_hbm, emb_hbm, cemb_hbm, pos_hbm, nout_hbm,
             tgt_idx, ctx_idx, neg_idx, tgt_rows, ctx_rows, neg_rows,
             pos_buf, neg_buf, sem):
    wid = lax.axis_index("s") * _NC + lax.axis_index("c")
    base = wid * _BW
    pltpu.sync_copy(tgt_hbm.at[pl.ds(base, _BW)], tgt_idx)
    pltpu.sync_copy(ctx_hbm.at[pl.ds(base, _BW)], ctx_idx)
    pltpu.sync_copy(neg_hbm.at[pl.ds(base * _K, _BW * _K)], neg_idx)
    lane = lax.iota(jnp.int32, 16)

    def chunk_body(c, carry):
        co = c * _CB
        handles = [
            pltpu.async_copy(emb_hbm.at[tgt_idx.at[pl.ds(co, _CB)]],
                             tgt_rows, sem),
            pltpu.async_copy(cemb_hbm.at[ctx_idx.at[pl.ds(co, _CB)]],
                             ctx_rows, sem),
        ]
        for g in range(_NEG_G):
            handles.append(pltpu.async_copy(
                cemb_hbm.at[neg_idx.at[pl.ds(co * _K + g * _GI, _GI)]],
                neg_rows.at[pl.ds(g * _GI, _GI), :], sem))
        for h in handles:
            h.wait()

        for g in range(_CB // 16):
            def item_body(i, accs):
                row = g * 16 + i
                t = [tgt_rows[row, pl.ds(q * 16, 16)] for q in range(4)]
                cx = [ctx_rows[row, pl.ds(q * 16, 16)] for q in range(4)]
                m = lane == i
                acc = (t[0] * cx[0] + t[1] * cx[1]) + (t[2] * cx[2] + t[3] * cx[3])
                out = [jnp.where(m, jnp.sum(acc), accs[0])]
                for k in range(_K):
                    r = row * _K + k
                    n = [neg_rows[r, pl.ds(q * 16, 16)] for q in range(4)]
                    acc = (t[0] * n[0] + t[1] * n[1]) + (t[2] * n[2] + t[3] * n[3])
                    out.append(jnp.where(m, jnp.sum(acc), accs[1 + k]))
                return tuple(out)

            zero = jnp.zeros((16,), jnp.float32)
            accs = lax.fori_loop(0, 16, item_body, (zero,) * (_K + 1))
            pos_buf[pl.ds(co + g * 16, 16)] = accs[0]
            for k in range(_K):
                neg_buf[k, pl.ds(co + g * 16, 16)] = accs[1 + k]
        return carry

    lax.fori_loop(0, _NCHUNK, chunk_body, 0)
    pltpu.sync_copy(pos_buf, pos_hbm.at[pl.ds(base, _BW)])
    pltpu.sync_copy(neg_buf, nout_hbm.at[wid])


@functools.cache
def _sc_dots():
    return pl.kernel(
        _sc_body,
        out_type=(jax.ShapeDtypeStruct((_B,), jnp.float32),
                  jax.ShapeDtypeStruct((_NW, _K, _BW), jnp.float32)),
        mesh=plsc.VectorSubcoreMesh(core_axis_name="c", subcore_axis_name="s",
                                    num_cores=_NC, num_subcores=_NS),
        compiler_params=pltpu.CompilerParams(needs_layout_passes=False,
                                             use_tc_tiling_on_sc=False),
        scratch_types=[
            pltpu.VMEM((_BW,), jnp.int32),
            pltpu.VMEM((_BW,), jnp.int32),
            pltpu.VMEM((_BW * _K,), jnp.int32),
            pltpu.VMEM((_CB, _D), jnp.float32),
            pltpu.VMEM((_CB, _D), jnp.float32),
            pltpu.VMEM((_NEG_I, _D), jnp.float32),
            pltpu.VMEM((_BW,), jnp.float32),
            pltpu.VMEM((_K, _BW), jnp.float32),
            pltpu.SemaphoreType.DMA,
        ],
    )


def _loss_body(pos_ref, neg_ref, out_ref):
    pos = pos_ref[...]
    neg = -neg_ref[...]
    ls_pos = jnp.minimum(pos, 0.0) - jnp.log1p(jnp.exp(-jnp.abs(pos)))
    ls_neg = jnp.minimum(neg, 0.0) - jnp.log1p(jnp.exp(-jnp.abs(neg)))
    sp = jnp.sum(ls_pos, axis=0, keepdims=True)
    sn = jnp.sum(ls_neg, axis=0, keepdims=True)
    out_ref[0, 0] = -jnp.sum(sp + sn) / _B


_loss_call = pl.pallas_call(
    _loss_body,
    out_shape=jax.ShapeDtypeStruct((1, 1), jnp.float32),
    out_specs=pl.BlockSpec(memory_space=pltpu.SMEM),
)


def kernel(target, context, negative_samples, emb, ctx_emb):
    tgt = target.astype(jnp.int32)
    ctx = context.astype(jnp.int32)
    neg = negative_samples.astype(jnp.int32).reshape(_B * _K)
    pos_d, neg_d = _sc_dots()(tgt, ctx, neg, emb, ctx_emb)
    loss = _loss_call(pos_d.reshape(128, 128), neg_d.reshape(_B * _K // 128, 128))
    return loss[0, 0]
